# baseline (speedup 1.0000x reference)
import functools

import jax
import jax.numpy as jnp
from jax import lax
from jax.experimental import pallas as pl
from jax.experimental.pallas import tpu as pltpu
from jax.experimental.pallas import tpu_sc as plsc

B = 16384
D = 32
V = 1000000
L = 16
NC = 2
NS = 16
NW = NC * NS
BPW = B // NW
CHUNK = 128
NCHUNK = BPW // CHUNK
RPL = 128 // D
NLINES = V * D // 128
WBLK = 512
NSTEP = -(-V // WBLK)

_mesh = plsc.VectorSubcoreMesh(core_axis_name="c", subcore_axis_name="s")


def _t_body(t_ref, o_ref):
    x = t_ref[...]
    eye = jnp.eye(D, dtype=jnp.float32)
    o_ref[...] = jnp.concatenate(
        [jax.lax.dot_general(
            x[:, q * CHUNK:(q + 1) * CHUNK], eye,
            (((0,), (0,)), ((), ())),
            preferred_element_type=jnp.float32)
         for q in range(RPL)],
        axis=1,
    )


_to_rows = pl.pallas_call(
    _t_body,
    out_shape=jax.ShapeDtypeStruct((NSTEP * CHUNK, 128), jnp.float32),
    grid=(NSTEP,),
    in_specs=[pl.BlockSpec((D, WBLK), lambda j: (0, j))],
    out_specs=pl.BlockSpec((CHUNK, CHUNK), lambda j: (j, 0)),
)


@functools.partial(
    pl.kernel,
    mesh=_mesh,
    out_type=jax.ShapeDtypeStruct((B,), jnp.float32),
    scratch_types=[
        pltpu.VMEM((NCHUNK, CHUNK), jnp.int32),
        pltpu.VMEM((NCHUNK, CHUNK), jnp.int32),
        pltpu.VMEM((NCHUNK, CHUNK), jnp.int32),
        pltpu.VMEM((NCHUNK, CHUNK), jnp.int32),
        pltpu.VMEM((2, CHUNK, 128), jnp.float32),
        pltpu.VMEM((2, CHUNK, 128), jnp.float32),
        pltpu.VMEM((BPW,), jnp.float32),
        pltpu.SemaphoreType.DMA,
    ],
)
def _w2v_kernel(cw_hbm, xw_hbm, ctab_hbm, xtab_hbm, out_hbm,
                cw_v, xw_v, cl_v, xl_v, cr_v, xr_v, o_v, sem):
    wid = lax.axis_index("s") * NC + lax.axis_index("c")
    base_chunk = wid * NCHUNK

    pltpu.sync_copy(cw_hbm.at[pl.ds(base_chunk, NCHUNK)], cw_v)
    pltpu.sync_copy(xw_hbm.at[pl.ds(base_chunk, NCHUNK)], xw_v)

    def line_of(w):
        hi = jax.lax.shift_right_logical(w, 9)
        return jax.lax.shift_left(hi, 7) | (w & 127)

    for c in range(NCHUNK):
        for g in range(CHUNK // L):
            sl = pl.ds(g * L, L)
            cl_v[c, sl] = line_of(cw_v[c, sl])
            xl_v[c, sl] = line_of(xw_v[c, sl])

    def fetch(c, buf):
        return (
            pltpu.async_copy(ctab_hbm.at[cl_v.at[c]], cr_v.at[buf], sem),
            pltpu.async_copy(xtab_hbm.at[xl_v.at[c]], xr_v.at[buf], sem),
        )

    lane = lax.iota(jnp.int32, L)
    perms = [lane ^ k for k in (8, 4, 2, 1)]

    def hsum(v):
        for p in perms:
            v = v + v.at[p].get(mode="promise_in_bounds")
        return v

    pend = fetch(0, 0)
    for c in range(NCHUNK):
        for cp in pend:
            cp.wait()
        if c + 1 < NCHUNK:
            pend = fetch(c + 1, (c + 1) % 2)
        buf = c % 2

        def body(g, carry, c=c, buf=buf):
            base = g * L
            ocv = (jax.lax.shift_right_logical(cw_v[c, pl.ds(base, L)], 7)
                   & (RPL - 1)) * D
            oxv = (jax.lax.shift_right_logical(xw_v[c, pl.ds(base, L)], 7)
                   & (RPL - 1)) * D
            out = jnp.zeros((L,), jnp.float32)
            for i in range(L):
                j = base + i
                oc = ocv[i]
                ox = oxv[i]
                c0 = cr_v[buf, j, pl.ds(oc, L)]
                c1 = cr_v[buf, j, pl.ds(oc + L, L)]
                x0 = xr_v[buf, j, pl.ds(ox, L)]
                x1 = xr_v[buf, j, pl.ds(ox + L, L)]
                s = c0 * x0 + c1 * x1
                out = jnp.where(lane == i, hsum(s), out)
            o_v[pl.ds(c * CHUNK + base, L)] = 1.0 / (1.0 + jnp.exp(-out))
            return carry

        lax.fori_loop(0, CHUNK // L, body, 0)

    pltpu.sync_copy(o_v, out_hbm.at[pl.ds(wid * BPW, BPW)])


def kernel(center_word, context_word, center_table, context_table):
    cw = center_word.astype(jnp.int32).reshape(B // CHUNK, CHUNK)
    xw = context_word.astype(jnp.int32).reshape(B // CHUNK, CHUNK)
    ct = _to_rows(center_table.T)
    xt = _to_rows(context_table.T)
    return _w2v_kernel(cw, xw, ct, xt)


# speedup vs baseline: 4.0662x; 4.0662x over previous
import functools

import jax
import jax.numpy as jnp
from jax import lax
from jax.experimental import pallas as pl
from jax.experimental.pallas import tpu as pltpu
from jax.experimental.pallas import tpu_sc as plsc

B = 16384
D = 32
V = 1000000
L = 16
NC = 2
NS = 16
NW = NC * NS
BPW = B // NW
CHUNK = 128
NCHUNK = BPW // CHUNK
RPL = 128 // D
NLINES = V * D // 128
WBLK = 4096
NSTEP = -(-V // WBLK)
SUB = WBLK // 512

_mesh = plsc.VectorSubcoreMesh(core_axis_name="c", subcore_axis_name="s")


def _t_body(t_ref, o_ref):
    x = t_ref[...]
    o_ref[...] = jnp.concatenate(
        [jnp.concatenate(
            [jnp.transpose(x[:, (s * RPL + q) * CHUNK:(s * RPL + q + 1) * CHUNK])
             for q in range(RPL)], axis=1)
         for s in range(SUB)], axis=0)


_to_rows = pl.pallas_call(
    _t_body,
    out_shape=jax.ShapeDtypeStruct((NSTEP * SUB * CHUNK, 128), jnp.float32),
    grid=(NSTEP,),
    in_specs=[pl.BlockSpec((D, WBLK), lambda j: (0, j))],
    out_specs=pl.BlockSpec((SUB * CHUNK, CHUNK), lambda j: (j, 0)),
)


@functools.partial(
    pl.kernel,
    mesh=_mesh,
    out_type=jax.ShapeDtypeStruct((B,), jnp.float32),
    scratch_types=[
        pltpu.VMEM((NCHUNK, CHUNK), jnp.int32),
        pltpu.VMEM((NCHUNK, CHUNK), jnp.int32),
        pltpu.VMEM((NCHUNK, CHUNK), jnp.int32),
        pltpu.VMEM((NCHUNK, CHUNK), jnp.int32),
        pltpu.VMEM((2, CHUNK, 128), jnp.float32),
        pltpu.VMEM((2, CHUNK, 128), jnp.float32),
        pltpu.VMEM((BPW,), jnp.float32),
        pltpu.SemaphoreType.DMA,
    ],
)
def _w2v_kernel(cw_hbm, xw_hbm, ctab_hbm, xtab_hbm, out_hbm,
                cw_v, xw_v, cl_v, xl_v, cr_v, xr_v, o_v, sem):
    wid = lax.axis_index("s") * NC + lax.axis_index("c")
    base_chunk = wid * NCHUNK

    pltpu.sync_copy(cw_hbm.at[pl.ds(base_chunk, NCHUNK)], cw_v)
    pltpu.sync_copy(xw_hbm.at[pl.ds(base_chunk, NCHUNK)], xw_v)

    def line_of(w):
        hi = jax.lax.shift_right_logical(w, 9)
        return jax.lax.shift_left(hi, 7) | (w & 127)

    for c in range(NCHUNK):
        for g in range(CHUNK // L):
            sl = pl.ds(g * L, L)
            cl_v[c, sl] = line_of(cw_v[c, sl])
            xl_v[c, sl] = line_of(xw_v[c, sl])

    def fetch(c, buf):
        return (
            pltpu.async_copy(ctab_hbm.at[cl_v.at[c]], cr_v.at[buf], sem),
            pltpu.async_copy(xtab_hbm.at[xl_v.at[c]], xr_v.at[buf], sem),
        )

    lane = lax.iota(jnp.int32, L)
    perms = [lane ^ k for k in (8, 4, 2, 1)]

    def hsum(v):
        for p in perms:
            v = v + v.at[p].get(mode="promise_in_bounds")
        return v

    pend = fetch(0, 0)
    for c in range(NCHUNK):
        for cp in pend:
            cp.wait()
        if c + 1 < NCHUNK:
            pend = fetch(c + 1, (c + 1) % 2)
        buf = c % 2

        def body(g, carry, c=c, buf=buf):
            base = g * L
            ocv = (jax.lax.shift_right_logical(cw_v[c, pl.ds(base, L)], 7)
                   & (RPL - 1)) * D
            oxv = (jax.lax.shift_right_logical(xw_v[c, pl.ds(base, L)], 7)
                   & (RPL - 1)) * D
            out = jnp.zeros((L,), jnp.float32)
            for i in range(L):
                j = base + i
                oc = ocv[i]
                ox = oxv[i]
                c0 = cr_v[buf, j, pl.ds(oc, L)]
                c1 = cr_v[buf, j, pl.ds(oc + L, L)]
                x0 = xr_v[buf, j, pl.ds(ox, L)]
                x1 = xr_v[buf, j, pl.ds(ox + L, L)]
                s = c0 * x0 + c1 * x1
                out = jnp.where(lane == i, hsum(s), out)
            o_v[pl.ds(c * CHUNK + base, L)] = 1.0 / (1.0 + jnp.exp(-out))
            return carry

        lax.fori_loop(0, CHUNK // L, body, 0)

    pltpu.sync_copy(o_v, out_hbm.at[pl.ds(wid * BPW, BPW)])


def kernel(center_word, context_word, center_table, context_table):
    cw = center_word.astype(jnp.int32).reshape(B // CHUNK, CHUNK)
    xw = context_word.astype(jnp.int32).reshape(B // CHUNK, CHUNK)
    ct = _to_rows(center_table.T)
    xt = _to_rows(context_table.T)
    return _w2v_kernel(cw, xw, ct, xt)


# speedup vs baseline: 4.8413x; 1.1906x over previous
import functools

import jax
import jax.numpy as jnp
from jax import lax
from jax.experimental import pallas as pl
from jax.experimental.pallas import tpu as pltpu
from jax.experimental.pallas import tpu_sc as plsc

B = 16384
D = 32
V = 1000000
L = 16
NC = 2
NS = 16
NW = NC * NS
BPW = B // NW
CHUNK = 128
NCHUNK = BPW // CHUNK
RPL = 128 // D
NLINES = V * D // 128
WBLK = 16384
NSTEP = -(-V // WBLK)
SUB = WBLK // 512

_mesh = plsc.VectorSubcoreMesh(core_axis_name="c", subcore_axis_name="s")


def _t_body(t_ref, o_ref):
    x = t_ref[...]
    o_ref[...] = jnp.concatenate(
        [jnp.concatenate(
            [jnp.transpose(x[:, (s * RPL + q) * CHUNK:(s * RPL + q + 1) * CHUNK])
             for q in range(RPL)], axis=1)
         for s in range(SUB)], axis=0)


_to_rows = pl.pallas_call(
    _t_body,
    out_shape=jax.ShapeDtypeStruct((NSTEP * SUB * CHUNK, 128), jnp.float32),
    grid=(NSTEP,),
    in_specs=[pl.BlockSpec((D, WBLK), lambda j: (0, j))],
    out_specs=pl.BlockSpec((SUB * CHUNK, CHUNK), lambda j: (j, 0)),
)


@functools.partial(
    pl.kernel,
    mesh=_mesh,
    out_type=jax.ShapeDtypeStruct((B,), jnp.float32),
    scratch_types=[
        pltpu.VMEM((NCHUNK, CHUNK), jnp.int32),
        pltpu.VMEM((NCHUNK, CHUNK), jnp.int32),
        pltpu.VMEM((NCHUNK, CHUNK), jnp.int32),
        pltpu.VMEM((NCHUNK, CHUNK), jnp.int32),
        pltpu.VMEM((2, CHUNK, 128), jnp.float32),
        pltpu.VMEM((2, CHUNK, 128), jnp.float32),
        pltpu.VMEM((BPW,), jnp.float32),
        pltpu.SemaphoreType.DMA,
    ],
)
def _w2v_kernel(cw_hbm, xw_hbm, ctab_hbm, xtab_hbm, out_hbm,
                cw_v, xw_v, cl_v, xl_v, cr_v, xr_v, o_v, sem):
    wid = lax.axis_index("s") * NC + lax.axis_index("c")
    base_chunk = wid * NCHUNK

    pltpu.sync_copy(cw_hbm.at[pl.ds(base_chunk, NCHUNK)], cw_v)
    pltpu.sync_copy(xw_hbm.at[pl.ds(base_chunk, NCHUNK)], xw_v)

    def line_of(w):
        hi = jax.lax.shift_right_logical(w, 9)
        return jax.lax.shift_left(hi, 7) | (w & 127)

    for c in range(NCHUNK):
        for g in range(CHUNK // L):
            sl = pl.ds(g * L, L)
            cl_v[c, sl] = line_of(cw_v[c, sl])
            xl_v[c, sl] = line_of(xw_v[c, sl])

    def fetch(c, buf):
        return (
            pltpu.async_copy(ctab_hbm.at[cl_v.at[c]], cr_v.at[buf], sem),
            pltpu.async_copy(xtab_hbm.at[xl_v.at[c]], xr_v.at[buf], sem),
        )

    lane = lax.iota(jnp.int32, L)
    perms = [lane ^ k for k in (8, 4, 2, 1)]

    def hsum(v):
        for p in perms:
            v = v + v.at[p].get(mode="promise_in_bounds")
        return v

    pend = fetch(0, 0)
    for c in range(NCHUNK):
        for cp in pend:
            cp.wait()
        if c + 1 < NCHUNK:
            pend = fetch(c + 1, (c + 1) % 2)
        buf = c % 2

        def body(g, carry, c=c, buf=buf):
            base = g * L
            ocv = (jax.lax.shift_right_logical(cw_v[c, pl.ds(base, L)], 7)
                   & (RPL - 1)) * D
            oxv = (jax.lax.shift_right_logical(xw_v[c, pl.ds(base, L)], 7)
                   & (RPL - 1)) * D
            out = jnp.zeros((L,), jnp.float32)
            for i in range(L):
                j = base + i
                oc = ocv[i]
                ox = oxv[i]
                c0 = cr_v[buf, j, pl.ds(oc, L)]
                c1 = cr_v[buf, j, pl.ds(oc + L, L)]
                x0 = xr_v[buf, j, pl.ds(ox, L)]
                x1 = xr_v[buf, j, pl.ds(ox + L, L)]
                s = c0 * x0 + c1 * x1
                out = jnp.where(lane == i, hsum(s), out)
            o_v[pl.ds(c * CHUNK + base, L)] = 1.0 / (1.0 + jnp.exp(-out))
            return carry

        lax.fori_loop(0, CHUNK // L, body, 0)

    pltpu.sync_copy(o_v, out_hbm.at[pl.ds(wid * BPW, BPW)])


def kernel(center_word, context_word, center_table, context_table):
    cw = center_word.astype(jnp.int32).reshape(B // CHUNK, CHUNK)
    xw = context_word.astype(jnp.int32).reshape(B // CHUNK, CHUNK)
    ct = _to_rows(center_table.T)
    xt = _to_rows(context_table.T)
    return _w2v_kernel(cw, xw, ct, xt)


# speedup vs baseline: 4.8867x; 1.0094x over previous
import functools

import jax
import jax.numpy as jnp
from jax import lax
from jax.experimental import pallas as pl
from jax.experimental.pallas import tpu as pltpu
from jax.experimental.pallas import tpu_sc as plsc

B = 16384
D = 32
V = 1000000
L = 16
NC = 2
NS = 16
NW = NC * NS
BPW = B // NW
CHUNK = 128
NCHUNK = BPW // CHUNK
RPL = 128 // D
NLINES = V * D // 128
WBLK = 32768
NSTEP = -(-V // WBLK)
SUB = WBLK // 512

_mesh = plsc.VectorSubcoreMesh(core_axis_name="c", subcore_axis_name="s")


def _t_body(t_ref, o_ref):
    x = t_ref[...]
    o_ref[...] = jnp.concatenate(
        [jnp.concatenate(
            [jnp.transpose(x[:, (s * RPL + q) * CHUNK:(s * RPL + q + 1) * CHUNK])
             for q in range(RPL)], axis=1)
         for s in range(SUB)], axis=0)


_to_rows = pl.pallas_call(
    _t_body,
    out_shape=jax.ShapeDtypeStruct((NSTEP * SUB * CHUNK, 128), jnp.float32),
    grid=(NSTEP,),
    in_specs=[pl.BlockSpec((D, WBLK), lambda j: (0, j))],
    out_specs=pl.BlockSpec((SUB * CHUNK, CHUNK), lambda j: (j, 0)),
)


@functools.partial(
    pl.kernel,
    mesh=_mesh,
    out_type=jax.ShapeDtypeStruct((B,), jnp.float32),
    scratch_types=[
        pltpu.VMEM((NCHUNK, CHUNK), jnp.int32),
        pltpu.VMEM((NCHUNK, CHUNK), jnp.int32),
        pltpu.VMEM((NCHUNK, CHUNK), jnp.int32),
        pltpu.VMEM((NCHUNK, CHUNK), jnp.int32),
        pltpu.VMEM((2, CHUNK, 128), jnp.float32),
        pltpu.VMEM((2, CHUNK, 128), jnp.float32),
        pltpu.VMEM((BPW,), jnp.float32),
        pltpu.SemaphoreType.DMA,
    ],
)
def _w2v_kernel(cw_hbm, xw_hbm, ctab_hbm, xtab_hbm, out_hbm,
                cw_v, xw_v, cl_v, xl_v, cr_v, xr_v, o_v, sem):
    wid = lax.axis_index("s") * NC + lax.axis_index("c")
    base_chunk = wid * NCHUNK

    pltpu.sync_copy(cw_hbm.at[pl.ds(base_chunk, NCHUNK)], cw_v)
    pltpu.sync_copy(xw_hbm.at[pl.ds(base_chunk, NCHUNK)], xw_v)

    def line_of(w):
        hi = jax.lax.shift_right_logical(w, 9)
        return jax.lax.shift_left(hi, 7) | (w & 127)

    for c in range(NCHUNK):
        for g in range(CHUNK // L):
            sl = pl.ds(g * L, L)
            cl_v[c, sl] = line_of(cw_v[c, sl])
            xl_v[c, sl] = line_of(xw_v[c, sl])

    def fetch(c, buf):
        return (
            pltpu.async_copy(ctab_hbm.at[cl_v.at[c]], cr_v.at[buf], sem),
            pltpu.async_copy(xtab_hbm.at[xl_v.at[c]], xr_v.at[buf], sem),
        )

    lane = lax.iota(jnp.int32, L)
    perms = [lane ^ k for k in (8, 4, 2, 1)]

    def hsum(v):
        for p in perms:
            v = v + v.at[p].get(mode="promise_in_bounds")
        return v

    pend = fetch(0, 0)
    for c in range(NCHUNK):
        for cp in pend:
            cp.wait()
        if c + 1 < NCHUNK:
            pend = fetch(c + 1, (c + 1) % 2)
        buf = c % 2

        def body(g, carry, c=c, buf=buf):
            base = g * L
            ocv = (jax.lax.shift_right_logical(cw_v[c, pl.ds(base, L)], 7)
                   & (RPL - 1)) * D
            oxv = (jax.lax.shift_right_logical(xw_v[c, pl.ds(base, L)], 7)
                   & (RPL - 1)) * D
            out = jnp.zeros((L,), jnp.float32)
            for i in range(L):
                j = base + i
                oc = ocv[i]
                ox = oxv[i]
                c0 = cr_v[buf, j, pl.ds(oc, L)]
                c1 = cr_v[buf, j, pl.ds(oc + L, L)]
                x0 = xr_v[buf, j, pl.ds(ox, L)]
                x1 = xr_v[buf, j, pl.ds(ox + L, L)]
                s = c0 * x0 + c1 * x1
                out = jnp.where(lane == i, hsum(s), out)
            o_v[pl.ds(c * CHUNK + base, L)] = 1.0 / (1.0 + jnp.exp(-out))
            return carry

        lax.fori_loop(0, CHUNK // L, body, 0)

    pltpu.sync_copy(o_v, out_hbm.at[pl.ds(wid * BPW, BPW)])


def kernel(center_word, context_word, center_table, context_table):
    cw = center_word.astype(jnp.int32).reshape(B // CHUNK, CHUNK)
    xw = context_word.astype(jnp.int32).reshape(B // CHUNK, CHUNK)
    ct = _to_rows(center_table.T)
    xt = _to_rows(context_table.T)
    return _w2v_kernel(cw, xw, ct, xt)
